# baseline (speedup 1.0000x reference)
import functools

import jax
import jax.numpy as jnp
from jax import lax
from jax.experimental import pallas as pl
from jax.experimental.pallas import tpu as pltpu
from jax.experimental.pallas import tpu_sc as plsc

_NC = 2
_NS = 16
_NW = _NC * _NS

_CHUNK = 80
_ZROWS = 128


def _sc_degree(dst, n_nodes):
    e = dst.shape[0]
    epw = e // _NW
    nch = epw // _CHUNK
    zlen = 640
    zstride = 624
    mesh = plsc.VectorSubcoreMesh(
        core_axis_name="c", subcore_axis_name="s",
        num_cores=_NC, num_subcores=_NS)

    @functools.partial(
        pl.kernel,
        out_type=jax.ShapeDtypeStruct((_NC * n_nodes,), jnp.float32),
        mesh=mesh,
        scratch_types=[
            pltpu.VMEM((_CHUNK,), jnp.int32),
            pltpu.VMEM((_CHUNK,), jnp.float32),
            pltpu.VMEM((zlen,), jnp.float32),
            pltpu.VMEM_SHARED((n_nodes,), jnp.float32),
        ],
    )
    def k(dst_hbm, ones_hbm, zeros_hbm, out_hbm, didx, ones_v, zb, acc):
        cid = lax.axis_index("c")
        sid = lax.axis_index("s")
        wid = cid * _NS + sid
        pltpu.sync_copy(ones_hbm, ones_v)
        pltpu.sync_copy(zeros_hbm, zb)
        pltpu.sync_copy(zb, acc.at[pl.ds(sid * zstride, zlen)])
        plsc.subcore_barrier()

        def body(g, _):
            off = wid * epw + g * _CHUNK
            pltpu.sync_copy(dst_hbm.at[pl.ds(off, _CHUNK)], didx)
            pltpu.sync_copy(ones_v, acc.at[didx], add=True)
            return 0

        lax.fori_loop(0, nch, body, 0)
        plsc.subcore_barrier()
        pltpu.sync_copy(acc.at[pl.ds(sid * zstride, zlen)], zb)
        pltpu.sync_copy(zb, out_hbm.at[pl.ds(cid * n_nodes + sid * zstride,
                                             zlen)])

    ones = jnp.ones((_CHUNK,), jnp.float32)
    zeros = jnp.zeros((zlen,), jnp.float32)
    return k(dst, ones, zeros).reshape(_NC, n_nodes)


_EC = 96
_ECH = 108
_RING = 4
_FIRE = 3
_LAG = 1
_RB = 80


def _sc_edge_sum(u, src1, dst1):
    n, d = u.shape
    nch = _ECH
    ec = _EC
    epw = nch * ec
    rstride = 624
    nz = 640 // _RB
    mesh = plsc.VectorSubcoreMesh(
        core_axis_name="c", subcore_axis_name="s",
        num_cores=_NC, num_subcores=_NS)

    @functools.partial(
        pl.kernel,
        out_type=jax.ShapeDtypeStruct((_NC, n, d), jnp.float32),
        mesh=mesh,
        scratch_types=[
            pltpu.VMEM((_RING, ec), jnp.int32),
            pltpu.VMEM((_RING, ec), jnp.int32),
            pltpu.VMEM((_RING, ec, d), jnp.float32),
            pltpu.VMEM_SHARED((n + 16, d), jnp.float32),
            pltpu.SemaphoreType.DMA,
            pltpu.SemaphoreType.DMA,
            pltpu.SemaphoreType.DMA,
        ],
    )
    def k(u_hbm, src_hbm, dst_hbm, zeros_hbm, out_hbm, sidx, didx, rows,
          acc, isem, gsem, ssem):
        cid = lax.axis_index("c")
        sid = lax.axis_index("s")
        wid = cid * _NS + sid
        zb = rows.at[0, pl.ds(0, _RB)]
        pltpu.sync_copy(zeros_hbm, zb)
        for j in range(nz):
            pltpu.sync_copy(zb, acc.at[pl.ds(sid * rstride + j * _RB, _RB)])

        def fire_idx(g, slot):
            off = wid * epw + g * ec
            pltpu.async_copy(src_hbm.at[pl.ds(off, ec)], sidx.at[slot], isem)
            pltpu.async_copy(dst_hbm.at[pl.ds(off, ec)], didx.at[slot], isem)

        def wait_idx(slot):
            pltpu.make_async_copy(src_hbm.at[pl.ds(0, ec)], sidx.at[slot],
                                  isem).wait()
            pltpu.make_async_copy(dst_hbm.at[pl.ds(0, ec)], didx.at[slot],
                                  isem).wait()

        def fire_gather(slot):
            pltpu.async_copy(u_hbm.at[sidx.at[slot]], rows.at[slot], gsem)

        def wait_gather(slot):
            pltpu.make_async_copy(u_hbm.at[sidx.at[0]], rows.at[slot],
                                  gsem).wait()

        def fire_scatter(slot):
            pltpu.async_copy(rows.at[slot], acc.at[didx.at[slot]], ssem,
                             add=True)

        def wait_scatter(slot):
            pltpu.make_async_copy(rows.at[slot], acc.at[didx.at[0]],
                                  ssem).wait()

        plsc.subcore_barrier()
        for r in range(_FIRE):
            fire_idx(r, r)
        for r in range(_FIRE):
            wait_idx(r)
            fire_gather(r)

        def body(s, _):
            g0 = s * _RING
            for r in range(_RING):
                g = g0 + r
                wait_gather(r)
                fire_scatter(r)

                @pl.when(g >= _LAG)
                def _():
                    wait_scatter((r - _LAG) % _RING)

                @pl.when(g + _FIRE < nch)
                def _():
                    nslot = (r + _FIRE) % _RING
                    fire_idx(g + _FIRE, nslot)
                    wait_idx(nslot)
                    fire_gather(nslot)

            return 0

        lax.fori_loop(0, nch // _RING, body, 0)
        for j in range(_LAG):
            wait_scatter((nch - _LAG + j) % _RING)
        plsc.subcore_barrier()
        for j in range(nz):
            r0 = sid * rstride + j * _RB
            pltpu.sync_copy(acc.at[pl.ds(r0, _RB)], zb)
            pltpu.sync_copy(zb, out_hbm.at[cid, pl.ds(r0, _RB)])

    zeros = jnp.zeros((_RB, d), jnp.float32)
    return k(u, src1, dst1, zeros)


_BR = 1000


def _tc_pre(h, w0, degp3):
    n, d = h.shape

    def body(h_ref, w_ref, dp_ref, u_ref, dis_ref):
        deg = dp_ref[0] + dp_ref[1]
        dis = jnp.where(deg > 0, lax.rsqrt(deg), 0.0)
        dis_ref[...] = dis
        u_ref[...] = jnp.dot(h_ref[...], w_ref[...],
                             preferred_element_type=jnp.float32) * dis

    return pl.pallas_call(
        body,
        grid=(n // _BR,),
        in_specs=[
            pl.BlockSpec((_BR, d), lambda i: (i, 0)),
            pl.BlockSpec((d, d), lambda i: (0, 0)),
            pl.BlockSpec((2, _BR, 1), lambda i: (0, i, 0)),
        ],
        out_specs=[
            pl.BlockSpec((_BR, d), lambda i: (i, 0)),
            pl.BlockSpec((_BR, 1), lambda i: (i, 0)),
        ],
        out_shape=[
            jax.ShapeDtypeStruct((n, d), jnp.float32),
            jax.ShapeDtypeStruct((n, 1), jnp.float32),
        ],
    )(h, w0, degp3)


def _ln(x, g, b):
    mu = jnp.mean(x, axis=-1, keepdims=True)
    xc = x - mu
    var = jnp.mean(xc * xc, axis=-1, keepdims=True)
    return xc * lax.rsqrt(var + 1e-5) * g + b


def _tc_mid0(p, dis, b0, g0, be0, w1):
    n, d = p.shape[1], p.shape[2]

    def body(p_ref, dis_ref, b_ref, g_ref, be_ref, w_ref, h_ref, u_ref):
        dis = dis_ref[...]
        s = (p_ref[0] + p_ref[1]) * dis + b_ref[...]
        hn = _ln(s, g_ref[...], be_ref[...])
        h_ref[...] = hn
        u_ref[...] = jnp.dot(hn, w_ref[...],
                             preferred_element_type=jnp.float32) * dis

    row = lambda i: (i, 0)
    fix = lambda i: (0, 0)
    return pl.pallas_call(
        body,
        grid=(n // _BR,),
        in_specs=[
            pl.BlockSpec((2, _BR, d), lambda i: (0, i, 0)),
            pl.BlockSpec((_BR, 1), row),
            pl.BlockSpec((1, d), fix),
            pl.BlockSpec((1, d), fix),
            pl.BlockSpec((1, d), fix),
            pl.BlockSpec((d, d), fix),
        ],
        out_specs=[pl.BlockSpec((_BR, d), row), pl.BlockSpec((_BR, d), row)],
        out_shape=[
            jax.ShapeDtypeStruct((n, d), jnp.float32),
            jax.ShapeDtypeStruct((n, d), jnp.float32),
        ],
    )(p, dis, b0, g0, be0, w1)


def _tc_mid1(p, dis, b1, g1, be1, hprev, ga, gb, gbias, w2):
    n, d = p.shape[1], p.shape[2]

    def body(p_ref, dis_ref, b_ref, g_ref, be_ref, hp_ref, ga_ref, gb_ref,
             gbias_ref, w_ref, h_ref, u_ref):
        dis = dis_ref[...]
        hp = hp_ref[...]
        s = (p_ref[0] + p_ref[1]) * dis + b_ref[...]
        hn = _ln(s, g_ref[...], be_ref[...])
        z = (jnp.dot(hp, ga_ref[...], preferred_element_type=jnp.float32)
             + jnp.dot(hn, gb_ref[...], preferred_element_type=jnp.float32)
             + gbias_ref[...])
        gate = jax.nn.sigmoid(z)
        hg = gate * hn + (1.0 - gate) * hp
        h_ref[...] = hg
        u_ref[...] = jnp.dot(hg, w_ref[...],
                             preferred_element_type=jnp.float32) * dis

    row = lambda i: (i, 0)
    fix = lambda i: (0, 0)
    return pl.pallas_call(
        body,
        grid=(n // _BR,),
        in_specs=[
            pl.BlockSpec((2, _BR, d), lambda i: (0, i, 0)),
            pl.BlockSpec((_BR, 1), row),
            pl.BlockSpec((1, d), fix),
            pl.BlockSpec((1, d), fix),
            pl.BlockSpec((1, d), fix),
            pl.BlockSpec((_BR, d), row),
            pl.BlockSpec((d, d), fix),
            pl.BlockSpec((d, d), fix),
            pl.BlockSpec((1, d), fix),
            pl.BlockSpec((d, d), fix),
        ],
        out_specs=[pl.BlockSpec((_BR, d), row), pl.BlockSpec((_BR, d), row)],
        out_shape=[
            jax.ShapeDtypeStruct((n, d), jnp.float32),
            jax.ShapeDtypeStruct((n, d), jnp.float32),
        ],
    )(p, dis, b1, g1, be1, hprev, ga, gb, gbias, w2)


def _tc_post(p, dis, b2, g2, be2, hprev, ga, gb, gbias, h_orig, res_w):
    n, d = p.shape[1], p.shape[2]

    def body(p_ref, dis_ref, b_ref, g_ref, be_ref, hp_ref, ga_ref, gb_ref,
             gbias_ref, ho_ref, rw_ref, dh_ref):
        hp = hp_ref[...]
        s = (p_ref[0] + p_ref[1]) * dis_ref[...] + b_ref[...]
        hn = _ln(s, g_ref[...], be_ref[...])
        z = (jnp.dot(hp, ga_ref[...], preferred_element_type=jnp.float32)
             + jnp.dot(hn, gb_ref[...], preferred_element_type=jnp.float32)
             + gbias_ref[...])
        gate = jax.nn.sigmoid(z)
        hg = gate * hn + (1.0 - gate) * hp
        dh_ref[...] = jnp.tanh(hg) + rw_ref[...] * ho_ref[...]

    row = lambda i: (i, 0)
    fix = lambda i: (0, 0)
    return pl.pallas_call(
        body,
        grid=(n // _BR,),
        in_specs=[
            pl.BlockSpec((2, _BR, d), lambda i: (0, i, 0)),
            pl.BlockSpec((_BR, 1), row),
            pl.BlockSpec((1, d), fix),
            pl.BlockSpec((1, d), fix),
            pl.BlockSpec((1, d), fix),
            pl.BlockSpec((_BR, d), row),
            pl.BlockSpec((d, d), fix),
            pl.BlockSpec((d, d), fix),
            pl.BlockSpec((1, d), fix),
            pl.BlockSpec((_BR, d), row),
            pl.BlockSpec((1, 1), fix),
        ],
        out_specs=pl.BlockSpec((_BR, d), row),
        out_shape=jax.ShapeDtypeStruct((n, d), jnp.float32),
    )(p, dis, b2, g2, be2, hprev, ga, gb, gbias, h_orig, res_w)


def kernel(t, h, edge_index, W0, b0, W1, b1, W2, b2, ln0_g, ln0_b, ln1_g,
           ln1_b, ln2_g, ln2_b, gate_W, gate_b, res_w):
    n, d = h.shape
    ei = edge_index.astype(jnp.int32)
    src = ei[0]
    dst = ei[1]
    e = src.shape[0]
    ep = _NW * _ECH * _EC
    pad = ep - e
    src1 = jnp.concatenate([src, jnp.zeros((pad,), jnp.int32)])
    dst1 = jnp.concatenate([dst, jnp.full((pad,), n, jnp.int32)])

    degp = _sc_degree(dst, n)
    degp3 = degp.reshape(_NC, n, 1)

    b0r = b0.reshape(1, d)
    b1r = b1.reshape(1, d)
    b2r = b2.reshape(1, d)
    g0 = ln0_g.reshape(1, d)
    be0 = ln0_b.reshape(1, d)
    g1 = ln1_g.reshape(1, d)
    be1 = ln1_b.reshape(1, d)
    g2 = ln2_g.reshape(1, d)
    be2 = ln2_b.reshape(1, d)
    ga = gate_W[:d]
    gb = gate_W[d:]
    gbias = gate_b.reshape(1, d)
    rw = res_w.reshape(1, 1)

    u0, dis = _tc_pre(h, W0, degp3)
    p0 = _sc_edge_sum(u0, src1, dst1)
    h1, u1 = _tc_mid0(p0, dis, b0r, g0, be0, W1)
    p1 = _sc_edge_sum(u1, src1, dst1)
    h2, u2 = _tc_mid1(p1, dis, b1r, g1, be1, h1, ga, gb, gbias, W2)
    p2 = _sc_edge_sum(u2, src1, dst1)
    dh = _tc_post(p2, dis, b2r, g2, be2, h2, ga, gb, gbias, h, rw)
    return dh


# speedup vs baseline: 1.6952x; 1.6952x over previous
import functools

import jax
import jax.numpy as jnp
from jax import lax
from jax.experimental import pallas as pl
from jax.experimental.pallas import tpu as pltpu
from jax.experimental.pallas import tpu_sc as plsc

_NC = 2
_NS = 16
_NW = _NC * _NS

_CHUNK = 80
_ZROWS = 128


def _sc_degree(dst, n_nodes):
    e = dst.shape[0]
    epw = e // _NW
    nch = epw // _CHUNK
    zlen = 640
    zstride = 624
    mesh = plsc.VectorSubcoreMesh(
        core_axis_name="c", subcore_axis_name="s",
        num_cores=_NC, num_subcores=_NS)

    @functools.partial(
        pl.kernel,
        out_type=jax.ShapeDtypeStruct((_NC * n_nodes,), jnp.float32),
        mesh=mesh,
        scratch_types=[
            pltpu.VMEM((_CHUNK,), jnp.int32),
            pltpu.VMEM((_CHUNK,), jnp.float32),
            pltpu.VMEM((zlen,), jnp.float32),
            pltpu.VMEM_SHARED((n_nodes,), jnp.float32),
        ],
    )
    def k(dst_hbm, ones_hbm, zeros_hbm, out_hbm, didx, ones_v, zb, acc):
        cid = lax.axis_index("c")
        sid = lax.axis_index("s")
        wid = cid * _NS + sid
        pltpu.sync_copy(ones_hbm, ones_v)
        pltpu.sync_copy(zeros_hbm, zb)
        pltpu.sync_copy(zb, acc.at[pl.ds(sid * zstride, zlen)])
        plsc.subcore_barrier()

        def body(g, _):
            off = wid * epw + g * _CHUNK
            pltpu.sync_copy(dst_hbm.at[pl.ds(off, _CHUNK)], didx)
            pltpu.sync_copy(ones_v, acc.at[didx], add=True)
            return 0

        lax.fori_loop(0, nch, body, 0)
        plsc.subcore_barrier()
        pltpu.sync_copy(acc.at[pl.ds(sid * zstride, zlen)], zb)
        pltpu.sync_copy(zb, out_hbm.at[pl.ds(cid * n_nodes + sid * zstride,
                                             zlen)])

    ones = jnp.ones((_CHUNK,), jnp.float32)
    zeros = jnp.zeros((zlen,), jnp.float32)
    return k(dst, ones, zeros).reshape(_NC, n_nodes)


_EC = 128
_ECH = 162
_RING = 6
_FIRE = 4
_LAG = 2
_RB = 80


def _sc_edge_sum(u3, src2, dst2):
    nc, n, dh = u3.shape
    ns, nch, ec = src2.shape
    rstride = 624
    nz = 640 // _RB
    mesh = plsc.VectorSubcoreMesh(
        core_axis_name="c", subcore_axis_name="s",
        num_cores=_NC, num_subcores=_NS)

    @functools.partial(
        pl.kernel,
        out_type=jax.ShapeDtypeStruct((_NC, n, dh), jnp.float32),
        mesh=mesh,
        scratch_types=[
            pltpu.VMEM((nch, ec), jnp.int32),
            pltpu.VMEM((nch, ec), jnp.int32),
            pltpu.VMEM((_RING, ec, dh), jnp.float32),
            pltpu.VMEM_SHARED((n + 16, dh), jnp.float32),
            pltpu.SemaphoreType.DMA,
            pltpu.SemaphoreType.DMA,
        ],
        compiler_params=pltpu.CompilerParams(use_tc_tiling_on_sc=False),
    )
    def k(u_hbm, src_hbm, dst_hbm, zeros_hbm, out_hbm, sidx, didx, rows,
          acc, gsem, ssem):
        cid = lax.axis_index("c")
        sid = lax.axis_index("s")
        uv = u_hbm.at[cid]
        zb = rows.at[0, pl.ds(0, _RB)]
        pltpu.sync_copy(zeros_hbm, zb)
        for j in range(nz):
            pltpu.sync_copy(zb, acc.at[pl.ds(sid * rstride + j * _RB, _RB)])
        pltpu.sync_copy(src_hbm.at[sid], sidx)
        pltpu.sync_copy(dst_hbm.at[sid], didx)

        def fire_gather(g, slot):
            pltpu.async_copy(uv.at[sidx.at[g]], rows.at[slot], gsem)

        def wait_gather(slot):
            pltpu.make_async_copy(uv.at[sidx.at[0]], rows.at[slot],
                                  gsem).wait()

        def fire_scatter(g, slot):
            pltpu.async_copy(rows.at[slot], acc.at[didx.at[g]], ssem,
                             add=True)

        def wait_scatter(slot):
            pltpu.make_async_copy(rows.at[slot], acc.at[didx.at[0]],
                                  ssem).wait()

        plsc.subcore_barrier()
        for r in range(_FIRE):
            fire_gather(r, r)

        def body(s, _):
            g0 = s * _RING
            for r in range(_RING):
                g = g0 + r
                wait_gather(r)
                fire_scatter(g, r)

                @pl.when(g >= _LAG)
                def _():
                    wait_scatter((r - _LAG) % _RING)

                @pl.when(g + _FIRE < nch)
                def _():
                    fire_gather(g + _FIRE, (r + _FIRE) % _RING)

            return 0

        lax.fori_loop(0, nch // _RING, body, 0)
        for j in range(_LAG):
            wait_scatter((nch - _LAG + j) % _RING)
        plsc.subcore_barrier()
        for j in range(nz):
            r0 = sid * rstride + j * _RB
            pltpu.sync_copy(acc.at[pl.ds(r0, _RB)], zb)
            pltpu.sync_copy(zb, out_hbm.at[cid, pl.ds(r0, _RB)])

    zeros = jnp.zeros((_RB, dh), jnp.float32)
    return k(u3, src2, dst2, zeros)


_BR = 1000


def _tc_pre(h, w0, degp3):
    n, d = h.shape

    def body(h_ref, w_ref, dp_ref, u_ref, dis_ref):
        deg = dp_ref[0] + dp_ref[1]
        dis = jnp.where(deg > 0, lax.rsqrt(deg), 0.0)
        dis_ref[...] = dis
        u = jnp.dot(h_ref[...], w_ref[...],
                    preferred_element_type=jnp.float32) * dis
        u_ref[0] = u[:, :d // 2]
        u_ref[1] = u[:, d // 2:]

    return pl.pallas_call(
        body,
        grid=(n // _BR,),
        in_specs=[
            pl.BlockSpec((_BR, d), lambda i: (i, 0)),
            pl.BlockSpec((d, d), lambda i: (0, 0)),
            pl.BlockSpec((2, _BR, 1), lambda i: (0, i, 0)),
        ],
        out_specs=[
            pl.BlockSpec((2, _BR, d // 2), lambda i: (0, i, 0)),
            pl.BlockSpec((_BR, 1), lambda i: (i, 0)),
        ],
        out_shape=[
            jax.ShapeDtypeStruct((2, n, d // 2), jnp.float32),
            jax.ShapeDtypeStruct((n, 1), jnp.float32),
        ],
    )(h, w0, degp3)


def _ln(x, g, b):
    mu = jnp.mean(x, axis=-1, keepdims=True)
    xc = x - mu
    var = jnp.mean(xc * xc, axis=-1, keepdims=True)
    return xc * lax.rsqrt(var + 1e-5) * g + b


def _tc_mid0(p, dis, b0, g0, be0, w1):
    n, d = p.shape[1], 2 * p.shape[2]

    def body(p_ref, dis_ref, b_ref, g_ref, be_ref, w_ref, h_ref, u_ref):
        dis = dis_ref[...]
        s = jnp.concatenate([p_ref[0], p_ref[1]], axis=-1) * dis + b_ref[...]
        hn = _ln(s, g_ref[...], be_ref[...])
        h_ref[...] = hn
        u = jnp.dot(hn, w_ref[...], preferred_element_type=jnp.float32) * dis
        u_ref[0] = u[:, :d // 2]
        u_ref[1] = u[:, d // 2:]

    row = lambda i: (i, 0)
    fix = lambda i: (0, 0)
    return pl.pallas_call(
        body,
        grid=(n // _BR,),
        in_specs=[
            pl.BlockSpec((2, _BR, d // 2), lambda i: (0, i, 0)),
            pl.BlockSpec((_BR, 1), row),
            pl.BlockSpec((1, d), fix),
            pl.BlockSpec((1, d), fix),
            pl.BlockSpec((1, d), fix),
            pl.BlockSpec((d, d), fix),
        ],
        out_specs=[pl.BlockSpec((_BR, d), row),
                   pl.BlockSpec((2, _BR, d // 2), lambda i: (0, i, 0))],
        out_shape=[
            jax.ShapeDtypeStruct((n, d), jnp.float32),
            jax.ShapeDtypeStruct((2, n, d // 2), jnp.float32),
        ],
    )(p, dis, b0, g0, be0, w1)


def _tc_mid1(p, dis, b1, g1, be1, hprev, ga, gb, gbias, w2):
    n, d = p.shape[1], 2 * p.shape[2]

    def body(p_ref, dis_ref, b_ref, g_ref, be_ref, hp_ref, ga_ref, gb_ref,
             gbias_ref, w_ref, h_ref, u_ref):
        dis = dis_ref[...]
        hp = hp_ref[...]
        s = jnp.concatenate([p_ref[0], p_ref[1]], axis=-1) * dis + b_ref[...]
        hn = _ln(s, g_ref[...], be_ref[...])
        z = (jnp.dot(hp, ga_ref[...], preferred_element_type=jnp.float32)
             + jnp.dot(hn, gb_ref[...], preferred_element_type=jnp.float32)
             + gbias_ref[...])
        gate = jax.nn.sigmoid(z)
        hg = gate * hn + (1.0 - gate) * hp
        h_ref[...] = hg
        u = jnp.dot(hg, w_ref[...], preferred_element_type=jnp.float32) * dis
        u_ref[0] = u[:, :d // 2]
        u_ref[1] = u[:, d // 2:]

    row = lambda i: (i, 0)
    fix = lambda i: (0, 0)
    return pl.pallas_call(
        body,
        grid=(n // _BR,),
        in_specs=[
            pl.BlockSpec((2, _BR, d // 2), lambda i: (0, i, 0)),
            pl.BlockSpec((_BR, 1), row),
            pl.BlockSpec((1, d), fix),
            pl.BlockSpec((1, d), fix),
            pl.BlockSpec((1, d), fix),
            pl.BlockSpec((_BR, d), row),
            pl.BlockSpec((d, d), fix),
            pl.BlockSpec((d, d), fix),
            pl.BlockSpec((1, d), fix),
            pl.BlockSpec((d, d), fix),
        ],
        out_specs=[pl.BlockSpec((_BR, d), row),
                   pl.BlockSpec((2, _BR, d // 2), lambda i: (0, i, 0))],
        out_shape=[
            jax.ShapeDtypeStruct((n, d), jnp.float32),
            jax.ShapeDtypeStruct((2, n, d // 2), jnp.float32),
        ],
    )(p, dis, b1, g1, be1, hprev, ga, gb, gbias, w2)


def _tc_post(p, dis, b2, g2, be2, hprev, ga, gb, gbias, h_orig, res_w):
    n, d = p.shape[1], 2 * p.shape[2]

    def body(p_ref, dis_ref, b_ref, g_ref, be_ref, hp_ref, ga_ref, gb_ref,
             gbias_ref, ho_ref, rw_ref, dh_ref):
        hp = hp_ref[...]
        s = (jnp.concatenate([p_ref[0], p_ref[1]], axis=-1) * dis_ref[...]
             + b_ref[...])
        hn = _ln(s, g_ref[...], be_ref[...])
        z = (jnp.dot(hp, ga_ref[...], preferred_element_type=jnp.float32)
             + jnp.dot(hn, gb_ref[...], preferred_element_type=jnp.float32)
             + gbias_ref[...])
        gate = jax.nn.sigmoid(z)
        hg = gate * hn + (1.0 - gate) * hp
        dh_ref[...] = jnp.tanh(hg) + rw_ref[...] * ho_ref[...]

    row = lambda i: (i, 0)
    fix = lambda i: (0, 0)
    return pl.pallas_call(
        body,
        grid=(n // _BR,),
        in_specs=[
            pl.BlockSpec((2, _BR, d // 2), lambda i: (0, i, 0)),
            pl.BlockSpec((_BR, 1), row),
            pl.BlockSpec((1, d), fix),
            pl.BlockSpec((1, d), fix),
            pl.BlockSpec((1, d), fix),
            pl.BlockSpec((_BR, d), row),
            pl.BlockSpec((d, d), fix),
            pl.BlockSpec((d, d), fix),
            pl.BlockSpec((1, d), fix),
            pl.BlockSpec((_BR, d), row),
            pl.BlockSpec((1, 1), fix),
        ],
        out_specs=pl.BlockSpec((_BR, d), row),
        out_shape=jax.ShapeDtypeStruct((n, d), jnp.float32),
    )(p, dis, b2, g2, be2, hprev, ga, gb, gbias, h_orig, res_w)


def kernel(t, h, edge_index, W0, b0, W1, b1, W2, b2, ln0_g, ln0_b, ln1_g,
           ln1_b, ln2_g, ln2_b, gate_W, gate_b, res_w):
    n, d = h.shape
    ei = edge_index.astype(jnp.int32)
    src = ei[0]
    dst = ei[1]
    e = src.shape[0]
    ep = _NS * _ECH * _EC
    pad = ep - e
    src2 = jnp.concatenate([src, jnp.zeros((pad,), jnp.int32)]
                           ).reshape(_NS, _ECH, _EC)
    dst2 = jnp.concatenate([dst, jnp.full((pad,), n, jnp.int32)]
                           ).reshape(_NS, _ECH, _EC)

    degp = _sc_degree(dst, n)
    degp3 = degp.reshape(_NC, n, 1)

    b0r = b0.reshape(1, d)
    b1r = b1.reshape(1, d)
    b2r = b2.reshape(1, d)
    g0 = ln0_g.reshape(1, d)
    be0 = ln0_b.reshape(1, d)
    g1 = ln1_g.reshape(1, d)
    be1 = ln1_b.reshape(1, d)
    g2 = ln2_g.reshape(1, d)
    be2 = ln2_b.reshape(1, d)
    ga = gate_W[:d]
    gb = gate_W[d:]
    gbias = gate_b.reshape(1, d)
    rw = res_w.reshape(1, 1)

    u0, dis = _tc_pre(h, W0, degp3)
    p0 = _sc_edge_sum(u0, src2, dst2)
    h1, u1 = _tc_mid0(p0, dis, b0r, g0, be0, W1)
    p1 = _sc_edge_sum(u1, src2, dst2)
    h2, u2 = _tc_mid1(p1, dis, b1r, g1, be1, h1, ga, gb, gbias, W2)
    p2 = _sc_edge_sum(u2, src2, dst2)
    dh = _tc_post(p2, dis, b2r, g2, be2, h2, ga, gb, gbias, h, rw)
    return dh


# speedup vs baseline: 3.1841x; 1.8783x over previous
import functools

import jax
import jax.numpy as jnp
from jax import lax
from jax.experimental import pallas as pl
from jax.experimental.pallas import tpu as pltpu
from jax.experimental.pallas import tpu_sc as plsc

_NC = 2
_NS = 16
_NW = _NC * _NS

_CHUNK = 80
_ZROWS = 128


def _sc_degree(dst, n_nodes):
    e = dst.shape[0]
    epw = e // _NW
    nch = epw // _CHUNK
    zlen = 640
    zstride = 624
    mesh = plsc.VectorSubcoreMesh(
        core_axis_name="c", subcore_axis_name="s",
        num_cores=_NC, num_subcores=_NS)

    @functools.partial(
        pl.kernel,
        out_type=jax.ShapeDtypeStruct((_NC * n_nodes,), jnp.float32),
        mesh=mesh,
        scratch_types=[
            pltpu.VMEM((_CHUNK,), jnp.int32),
            pltpu.VMEM((_CHUNK,), jnp.float32),
            pltpu.VMEM((zlen,), jnp.float32),
            pltpu.VMEM_SHARED((n_nodes,), jnp.float32),
        ],
    )
    def k(dst_hbm, ones_hbm, zeros_hbm, out_hbm, didx, ones_v, zb, acc):
        cid = lax.axis_index("c")
        sid = lax.axis_index("s")
        wid = cid * _NS + sid
        pltpu.sync_copy(ones_hbm, ones_v)
        pltpu.sync_copy(zeros_hbm, zb)
        pltpu.sync_copy(zb, acc.at[pl.ds(sid * zstride, zlen)])
        plsc.subcore_barrier()

        def body(g, _):
            off = wid * epw + g * _CHUNK
            pltpu.sync_copy(dst_hbm.at[pl.ds(off, _CHUNK)], didx)
            pltpu.sync_copy(ones_v, acc.at[didx], add=True)
            return 0

        lax.fori_loop(0, nch, body, 0)
        plsc.subcore_barrier()
        pltpu.sync_copy(acc.at[pl.ds(sid * zstride, zlen)], zb)
        pltpu.sync_copy(zb, out_hbm.at[pl.ds(cid * n_nodes + sid * zstride,
                                             zlen)])

    ones = jnp.ones((_CHUNK,), jnp.float32)
    zeros = jnp.zeros((zlen,), jnp.float32)
    return k(dst, ones, zeros).reshape(_NC, n_nodes)


_EC = 72
_ECH = 140
_RING = 5
_FIRE = 3
_LAG = 2
_IRING = 10
_IAHEAD = 6
_IUNROLL = 10
_RB = 80


def _sc_edge_sum(u, src1, dst1):
    n, d = u.shape
    nch = _ECH
    ec = _EC
    epw = nch * ec
    rstride = 624
    nz = 640 // _RB
    mesh = plsc.VectorSubcoreMesh(
        core_axis_name="c", subcore_axis_name="s",
        num_cores=_NC, num_subcores=_NS)

    @functools.partial(
        pl.kernel,
        out_type=jax.ShapeDtypeStruct((_NC, n, d), jnp.float32),
        mesh=mesh,
        scratch_types=[
            pltpu.VMEM((_IRING, ec), jnp.int32),
            pltpu.VMEM((_IRING, ec), jnp.int32),
            pltpu.VMEM((_RING, ec, d), jnp.float32),
            pltpu.VMEM_SHARED((n + 16, d), jnp.float32),
            pltpu.SemaphoreType.DMA,
            pltpu.SemaphoreType.DMA,
            pltpu.SemaphoreType.DMA,
        ],
    )
    def k(u_hbm, src_hbm, dst_hbm, zeros_hbm, out_hbm, sidx, didx, rows,
          acc, isem, gsem, ssem):
        cid = lax.axis_index("c")
        sid = lax.axis_index("s")
        wid = cid * _NS + sid
        zb = rows.at[0, pl.ds(0, _RB)]
        pltpu.sync_copy(zeros_hbm, zb)
        for j in range(nz):
            pltpu.sync_copy(zb, acc.at[pl.ds(sid * rstride + j * _RB, _RB)])

        def fire_idx(g, islot):
            off = wid * epw + g * ec
            pltpu.async_copy(src_hbm.at[pl.ds(off, ec)], sidx.at[islot],
                             isem)
            pltpu.async_copy(dst_hbm.at[pl.ds(off, ec)], didx.at[islot],
                             isem)

        def wait_idx():
            pltpu.make_async_copy(src_hbm.at[pl.ds(0, ec)], sidx.at[0],
                                  isem).wait()
            pltpu.make_async_copy(dst_hbm.at[pl.ds(0, ec)], didx.at[0],
                                  isem).wait()

        def fire_gather(islot, slot):
            pltpu.async_copy(u_hbm.at[sidx.at[islot]], rows.at[slot], gsem)

        def wait_gather(slot):
            pltpu.make_async_copy(u_hbm.at[sidx.at[0]], rows.at[slot],
                                  gsem).wait()

        def fire_scatter(islot, slot):
            pltpu.async_copy(rows.at[slot], acc.at[didx.at[islot]], ssem,
                             add=True)

        def wait_scatter(slot):
            pltpu.make_async_copy(rows.at[slot], acc.at[didx.at[0]],
                                  ssem).wait()

        plsc.subcore_barrier()
        for g in range(_IAHEAD):
            fire_idx(g, g)
        for g in range(_FIRE):
            wait_idx()
            fire_gather(g, g)

        def body(s, _):
            g0 = s * _IUNROLL
            for j in range(_IUNROLL):
                g = g0 + j
                r = j % _RING
                wait_gather(r)
                fire_scatter(j % _IRING, r)

                @pl.when(g >= _LAG)
                def _():
                    wait_scatter((r - _LAG) % _RING)

                @pl.when(g + _IAHEAD < nch)
                def _():
                    fire_idx(g + _IAHEAD, (j + _IAHEAD) % _IRING)

                @pl.when(g + _FIRE < nch)
                def _():
                    wait_idx()
                    fire_gather((j + _FIRE) % _IRING, (r + _FIRE) % _RING)

            return 0

        lax.fori_loop(0, nch // _IUNROLL, body, 0)
        for j in range(_LAG):
            wait_scatter((nch - _LAG + j) % _RING)
        plsc.subcore_barrier()
        for j in range(nz):
            r0 = sid * rstride + j * _RB
            pltpu.sync_copy(acc.at[pl.ds(r0, _RB)], zb)
            pltpu.sync_copy(zb, out_hbm.at[cid, pl.ds(r0, _RB)])

    zeros = jnp.zeros((_RB, d), jnp.float32)
    return k(u, src1, dst1, zeros)


_BR = 1000


def _tc_pre(h, w0, degp3):
    n, d = h.shape

    def body(h_ref, w_ref, dp_ref, u_ref, dis_ref):
        deg = dp_ref[0] + dp_ref[1]
        dis = jnp.where(deg > 0, lax.rsqrt(deg), 0.0)
        dis_ref[...] = dis
        u_ref[...] = jnp.dot(h_ref[...], w_ref[...],
                             preferred_element_type=jnp.float32) * dis

    return pl.pallas_call(
        body,
        grid=(n // _BR,),
        in_specs=[
            pl.BlockSpec((_BR, d), lambda i: (i, 0)),
            pl.BlockSpec((d, d), lambda i: (0, 0)),
            pl.BlockSpec((2, _BR, 1), lambda i: (0, i, 0)),
        ],
        out_specs=[
            pl.BlockSpec((_BR, d), lambda i: (i, 0)),
            pl.BlockSpec((_BR, 1), lambda i: (i, 0)),
        ],
        out_shape=[
            jax.ShapeDtypeStruct((n, d), jnp.float32),
            jax.ShapeDtypeStruct((n, 1), jnp.float32),
        ],
    )(h, w0, degp3)


def _ln(x, g, b):
    mu = jnp.mean(x, axis=-1, keepdims=True)
    xc = x - mu
    var = jnp.mean(xc * xc, axis=-1, keepdims=True)
    return xc * lax.rsqrt(var + 1e-5) * g + b


def _tc_mid0(p, dis, b0, g0, be0, w1):
    n, d = p.shape[1], p.shape[2]

    def body(p_ref, dis_ref, b_ref, g_ref, be_ref, w_ref, h_ref, u_ref):
        dis = dis_ref[...]
        s = (p_ref[0] + p_ref[1]) * dis + b_ref[...]
        hn = _ln(s, g_ref[...], be_ref[...])
        h_ref[...] = hn
        u_ref[...] = jnp.dot(hn, w_ref[...],
                             preferred_element_type=jnp.float32) * dis

    row = lambda i: (i, 0)
    fix = lambda i: (0, 0)
    return pl.pallas_call(
        body,
        grid=(n // _BR,),
        in_specs=[
            pl.BlockSpec((2, _BR, d), lambda i: (0, i, 0)),
            pl.BlockSpec((_BR, 1), row),
            pl.BlockSpec((1, d), fix),
            pl.BlockSpec((1, d), fix),
            pl.BlockSpec((1, d), fix),
            pl.BlockSpec((d, d), fix),
        ],
        out_specs=[pl.BlockSpec((_BR, d), row), pl.BlockSpec((_BR, d), row)],
        out_shape=[
            jax.ShapeDtypeStruct((n, d), jnp.float32),
            jax.ShapeDtypeStruct((n, d), jnp.float32),
        ],
    )(p, dis, b0, g0, be0, w1)


def _tc_mid1(p, dis, b1, g1, be1, hprev, ga, gb, gbias, w2):
    n, d = p.shape[1], p.shape[2]

    def body(p_ref, dis_ref, b_ref, g_ref, be_ref, hp_ref, ga_ref, gb_ref,
             gbias_ref, w_ref, h_ref, u_ref):
        dis = dis_ref[...]
        hp = hp_ref[...]
        s = (p_ref[0] + p_ref[1]) * dis + b_ref[...]
        hn = _ln(s, g_ref[...], be_ref[...])
        z = (jnp.dot(hp, ga_ref[...], preferred_element_type=jnp.float32)
             + jnp.dot(hn, gb_ref[...], preferred_element_type=jnp.float32)
             + gbias_ref[...])
        gate = jax.nn.sigmoid(z)
        hg = gate * hn + (1.0 - gate) * hp
        h_ref[...] = hg
        u_ref[...] = jnp.dot(hg, w_ref[...],
                             preferred_element_type=jnp.float32) * dis

    row = lambda i: (i, 0)
    fix = lambda i: (0, 0)
    return pl.pallas_call(
        body,
        grid=(n // _BR,),
        in_specs=[
            pl.BlockSpec((2, _BR, d), lambda i: (0, i, 0)),
            pl.BlockSpec((_BR, 1), row),
            pl.BlockSpec((1, d), fix),
            pl.BlockSpec((1, d), fix),
            pl.BlockSpec((1, d), fix),
            pl.BlockSpec((_BR, d), row),
            pl.BlockSpec((d, d), fix),
            pl.BlockSpec((d, d), fix),
            pl.BlockSpec((1, d), fix),
            pl.BlockSpec((d, d), fix),
        ],
        out_specs=[pl.BlockSpec((_BR, d), row), pl.BlockSpec((_BR, d), row)],
        out_shape=[
            jax.ShapeDtypeStruct((n, d), jnp.float32),
            jax.ShapeDtypeStruct((n, d), jnp.float32),
        ],
    )(p, dis, b1, g1, be1, hprev, ga, gb, gbias, w2)


def _tc_post(p, dis, b2, g2, be2, hprev, ga, gb, gbias, h_orig, res_w):
    n, d = p.shape[1], p.shape[2]

    def body(p_ref, dis_ref, b_ref, g_ref, be_ref, hp_ref, ga_ref, gb_ref,
             gbias_ref, ho_ref, rw_ref, dh_ref):
        hp = hp_ref[...]
        s = (p_ref[0] + p_ref[1]) * dis_ref[...] + b_ref[...]
        hn = _ln(s, g_ref[...], be_ref[...])
        z = (jnp.dot(hp, ga_ref[...], preferred_element_type=jnp.float32)
             + jnp.dot(hn, gb_ref[...], preferred_element_type=jnp.float32)
             + gbias_ref[...])
        gate = jax.nn.sigmoid(z)
        hg = gate * hn + (1.0 - gate) * hp
        dh_ref[...] = jnp.tanh(hg) + rw_ref[...] * ho_ref[...]

    row = lambda i: (i, 0)
    fix = lambda i: (0, 0)
    return pl.pallas_call(
        body,
        grid=(n // _BR,),
        in_specs=[
            pl.BlockSpec((2, _BR, d), lambda i: (0, i, 0)),
            pl.BlockSpec((_BR, 1), row),
            pl.BlockSpec((1, d), fix),
            pl.BlockSpec((1, d), fix),
            pl.BlockSpec((1, d), fix),
            pl.BlockSpec((_BR, d), row),
            pl.BlockSpec((d, d), fix),
            pl.BlockSpec((d, d), fix),
            pl.BlockSpec((1, d), fix),
            pl.BlockSpec((_BR, d), row),
            pl.BlockSpec((1, 1), fix),
        ],
        out_specs=pl.BlockSpec((_BR, d), row),
        out_shape=jax.ShapeDtypeStruct((n, d), jnp.float32),
    )(p, dis, b2, g2, be2, hprev, ga, gb, gbias, h_orig, res_w)


def kernel(t, h, edge_index, W0, b0, W1, b1, W2, b2, ln0_g, ln0_b, ln1_g,
           ln1_b, ln2_g, ln2_b, gate_W, gate_b, res_w):
    n, d = h.shape
    ei = edge_index.astype(jnp.int32)
    src = ei[0]
    dst = ei[1]
    e = src.shape[0]
    ep = _NW * _ECH * _EC
    pad = ep - e
    src1 = jnp.concatenate([src, jnp.zeros((pad,), jnp.int32)])
    dst1 = jnp.concatenate([dst, jnp.full((pad,), n, jnp.int32)])

    degp = _sc_degree(dst, n)
    degp3 = degp.reshape(_NC, n, 1)

    b0r = b0.reshape(1, d)
    b1r = b1.reshape(1, d)
    b2r = b2.reshape(1, d)
    g0 = ln0_g.reshape(1, d)
    be0 = ln0_b.reshape(1, d)
    g1 = ln1_g.reshape(1, d)
    be1 = ln1_b.reshape(1, d)
    g2 = ln2_g.reshape(1, d)
    be2 = ln2_b.reshape(1, d)
    ga = gate_W[:d]
    gb = gate_W[d:]
    gbias = gate_b.reshape(1, d)
    rw = res_w.reshape(1, 1)

    u0, dis = _tc_pre(h, W0, degp3)
    p0 = _sc_edge_sum(u0, src1, dst1)
    h1, u1 = _tc_mid0(p0, dis, b0r, g0, be0, W1)
    p1 = _sc_edge_sum(u1, src1, dst1)
    h2, u2 = _tc_mid1(p1, dis, b1r, g1, be1, h1, ga, gb, gbias, W2)
    p2 = _sc_edge_sum(u2, src1, dst1)
    dh = _tc_post(p2, dis, b2r, g2, be2, h2, ga, gb, gbias, h, rw)
    return dh


# speedup vs baseline: 3.3190x; 1.0424x over previous
import functools

import jax
import jax.numpy as jnp
from jax import lax
from jax.experimental import pallas as pl
from jax.experimental.pallas import tpu as pltpu
from jax.experimental.pallas import tpu_sc as plsc

_NC = 2
_NS = 16
_NW = _NC * _NS

_CHUNK = 80
_ZROWS = 128


def _sc_degree(dst, n_nodes):
    e = dst.shape[0]
    epw = e // _NW
    nch = epw // _CHUNK
    zlen = 640
    zstride = 624
    mesh = plsc.VectorSubcoreMesh(
        core_axis_name="c", subcore_axis_name="s",
        num_cores=_NC, num_subcores=_NS)

    @functools.partial(
        pl.kernel,
        out_type=jax.ShapeDtypeStruct((_NC * n_nodes,), jnp.float32),
        mesh=mesh,
        scratch_types=[
            pltpu.VMEM((_CHUNK,), jnp.int32),
            pltpu.VMEM((_CHUNK,), jnp.float32),
            pltpu.VMEM((zlen,), jnp.float32),
            pltpu.VMEM_SHARED((n_nodes,), jnp.float32),
        ],
    )
    def k(dst_hbm, ones_hbm, zeros_hbm, out_hbm, didx, ones_v, zb, acc):
        cid = lax.axis_index("c")
        sid = lax.axis_index("s")
        wid = cid * _NS + sid
        pltpu.sync_copy(ones_hbm, ones_v)
        pltpu.sync_copy(zeros_hbm, zb)
        pltpu.sync_copy(zb, acc.at[pl.ds(sid * zstride, zlen)])
        plsc.subcore_barrier()

        def body(g, _):
            off = wid * epw + g * _CHUNK
            pltpu.sync_copy(dst_hbm.at[pl.ds(off, _CHUNK)], didx)
            pltpu.sync_copy(ones_v, acc.at[didx], add=True)
            return 0

        lax.fori_loop(0, nch, body, 0)
        plsc.subcore_barrier()
        pltpu.sync_copy(acc.at[pl.ds(sid * zstride, zlen)], zb)
        pltpu.sync_copy(zb, out_hbm.at[pl.ds(cid * n_nodes + sid * zstride,
                                             zlen)])

    ones = jnp.ones((_CHUNK,), jnp.float32)
    zeros = jnp.zeros((zlen,), jnp.float32)
    return k(dst, ones, zeros).reshape(_NC, n_nodes)


_EC = 72
_ECH = 140
_RING = 5
_FIRE = 3
_LAG = 2
_IRING = 10
_IAHEAD = 6
_IUNROLL = 10
_RB = 80


def _sc_edge_sum(u, src1, dst1):
    n, d = u.shape
    nch = _ECH
    ec = _EC
    epw = nch * ec
    rstride = 624
    nz = 640 // _RB
    mesh = plsc.VectorSubcoreMesh(
        core_axis_name="c", subcore_axis_name="s",
        num_cores=_NC, num_subcores=_NS)

    @functools.partial(
        pl.kernel,
        out_type=jax.ShapeDtypeStruct((_NC, n, d), jnp.float32),
        mesh=mesh,
        scratch_types=[
            pltpu.VMEM((_IRING, ec), jnp.int32),
            pltpu.VMEM((_IRING, ec), jnp.int32),
            pltpu.VMEM((_RING, ec, d), jnp.float32),
            pltpu.VMEM_SHARED((n + 16, d), jnp.float32),
            pltpu.SemaphoreType.DMA,
            pltpu.SemaphoreType.DMA,
            pltpu.SemaphoreType.DMA,
        ],
    )
    def k(u_hbm, src_hbm, dst_hbm, zeros_hbm, out_hbm, sidx, didx, rows,
          acc, isem, gsem, ssem):
        cid = lax.axis_index("c")
        sid = lax.axis_index("s")
        wid = cid * _NS + sid
        zb = rows.at[0, pl.ds(0, _RB)]
        pltpu.sync_copy(zeros_hbm, zb)
        for j in range(nz):
            pltpu.sync_copy(zb, acc.at[pl.ds(sid * rstride + j * _RB, _RB)])

        def fire_idx(g, islot):
            off = wid * epw + g * ec
            pltpu.async_copy(src_hbm.at[pl.ds(off, ec)], sidx.at[islot],
                             isem)
            pltpu.async_copy(dst_hbm.at[pl.ds(off, ec)], didx.at[islot],
                             isem)

        def wait_idx():
            pltpu.make_async_copy(src_hbm.at[pl.ds(0, ec)], sidx.at[0],
                                  isem).wait()
            pltpu.make_async_copy(dst_hbm.at[pl.ds(0, ec)], didx.at[0],
                                  isem).wait()

        def fire_gather(islot, slot):
            pltpu.async_copy(u_hbm.at[sidx.at[islot]], rows.at[slot], gsem)

        def wait_gather(slot):
            pltpu.make_async_copy(u_hbm.at[sidx.at[0]], rows.at[slot],
                                  gsem).wait()

        def fire_scatter(islot, slot):
            pltpu.async_copy(rows.at[slot], acc.at[didx.at[islot]], ssem,
                             add=True)

        def wait_scatter(slot):
            pltpu.make_async_copy(rows.at[slot], acc.at[didx.at[0]],
                                  ssem).wait()

        plsc.subcore_barrier()
        for g in range(_IAHEAD):
            fire_idx(g, g)
        for g in range(_FIRE):
            wait_idx()
            fire_gather(g, g)

        def body(s, _):
            g0 = s * _IUNROLL
            for j in range(_IUNROLL):
                g = g0 + j
                r = j % _RING
                wait_gather(r)
                fire_scatter(j % _IRING, r)

                @pl.when(g >= _LAG)
                def _():
                    wait_scatter((r - _LAG) % _RING)

                @pl.when(g + _IAHEAD < nch)
                def _():
                    fire_idx(g + _IAHEAD, (j + _IAHEAD) % _IRING)

                @pl.when(g + _FIRE < nch)
                def _():
                    wait_idx()
                    fire_gather((j + _FIRE) % _IRING, (r + _FIRE) % _RING)

            return 0

        lax.fori_loop(0, nch // _IUNROLL, body, 0)
        for j in range(_LAG):
            wait_scatter((nch - _LAG + j) % _RING)
        plsc.subcore_barrier()
        for j in range(nz):
            r0 = sid * rstride + j * _RB
            pltpu.sync_copy(acc.at[pl.ds(r0, _RB)], zb)
            pltpu.sync_copy(zb, out_hbm.at[cid, pl.ds(r0, _RB)])

    zeros = jnp.zeros((_RB, d), jnp.float32)
    return k(u, src1, dst1, zeros)


_BR = 1000


def _tc_pre(h, w0, degp3):
    n, d = h.shape

    def body(h_ref, w_ref, dp_ref, u_ref, dis_ref):
        deg = dp_ref[0] + dp_ref[1]
        dis = jnp.where(deg > 0, lax.rsqrt(deg), 0.0)
        dis_ref[...] = dis
        u_ref[...] = jnp.dot(h_ref[...], w_ref[...],
                             preferred_element_type=jnp.float32) * dis

    return pl.pallas_call(
        body,
        grid=(n // _BR,),
        in_specs=[
            pl.BlockSpec((_BR, d), lambda i: (i, 0)),
            pl.BlockSpec((d, d), lambda i: (0, 0)),
            pl.BlockSpec((2, _BR, 1), lambda i: (0, i, 0)),
        ],
        out_specs=[
            pl.BlockSpec((_BR, d), lambda i: (i, 0)),
            pl.BlockSpec((_BR, 1), lambda i: (i, 0)),
        ],
        out_shape=[
            jax.ShapeDtypeStruct((n, d), jnp.float32),
            jax.ShapeDtypeStruct((n, 1), jnp.float32),
        ],
    )(h, w0, degp3)


def _ln(x, g, b):
    mu = jnp.mean(x, axis=-1, keepdims=True)
    xc = x - mu
    var = jnp.mean(xc * xc, axis=-1, keepdims=True)
    return xc * lax.rsqrt(var + 1e-5) * g + b


def _tc_mid0(p, dis, b0, g0, be0, w1):
    n, d = p.shape[1], p.shape[2]

    def body(p_ref, dis_ref, b_ref, g_ref, be_ref, w_ref, h_ref, u_ref):
        dis = dis_ref[...]
        s = (p_ref[0] + p_ref[1]) * dis + b_ref[...]
        hn = _ln(s, g_ref[...], be_ref[...])
        h_ref[...] = hn
        u_ref[...] = jnp.dot(hn, w_ref[...],
                             preferred_element_type=jnp.float32) * dis

    row = lambda i: (i, 0)
    fix = lambda i: (0, 0)
    return pl.pallas_call(
        body,
        grid=(n // _BR,),
        in_specs=[
            pl.BlockSpec((2, _BR, d), lambda i: (0, i, 0)),
            pl.BlockSpec((_BR, 1), row),
            pl.BlockSpec((1, d), fix),
            pl.BlockSpec((1, d), fix),
            pl.BlockSpec((1, d), fix),
            pl.BlockSpec((d, d), fix),
        ],
        out_specs=[pl.BlockSpec((_BR, d), row), pl.BlockSpec((_BR, d), row)],
        out_shape=[
            jax.ShapeDtypeStruct((n, d), jnp.float32),
            jax.ShapeDtypeStruct((n, d), jnp.float32),
        ],
    )(p, dis, b0, g0, be0, w1)


def _tc_mid1(p, dis, b1, g1, be1, hprev, ga, gb, gbias, w2):
    n, d = p.shape[1], p.shape[2]

    def body(p_ref, dis_ref, b_ref, g_ref, be_ref, hp_ref, ga_ref, gb_ref,
             gbias_ref, w_ref, h_ref, u_ref):
        dis = dis_ref[...]
        hp = hp_ref[...]
        s = (p_ref[0] + p_ref[1]) * dis + b_ref[...]
        hn = _ln(s, g_ref[...], be_ref[...])
        z = (jnp.dot(hp, ga_ref[...], preferred_element_type=jnp.float32)
             + jnp.dot(hn, gb_ref[...], preferred_element_type=jnp.float32)
             + gbias_ref[...])
        gate = jax.nn.sigmoid(z)
        hg = gate * hn + (1.0 - gate) * hp
        h_ref[...] = hg
        u_ref[...] = jnp.dot(hg, w_ref[...],
                             preferred_element_type=jnp.float32) * dis

    row = lambda i: (i, 0)
    fix = lambda i: (0, 0)
    return pl.pallas_call(
        body,
        grid=(n // _BR,),
        in_specs=[
            pl.BlockSpec((2, _BR, d), lambda i: (0, i, 0)),
            pl.BlockSpec((_BR, 1), row),
            pl.BlockSpec((1, d), fix),
            pl.BlockSpec((1, d), fix),
            pl.BlockSpec((1, d), fix),
            pl.BlockSpec((_BR, d), row),
            pl.BlockSpec((d, d), fix),
            pl.BlockSpec((d, d), fix),
            pl.BlockSpec((1, d), fix),
            pl.BlockSpec((d, d), fix),
        ],
        out_specs=[pl.BlockSpec((_BR, d), row), pl.BlockSpec((_BR, d), row)],
        out_shape=[
            jax.ShapeDtypeStruct((n, d), jnp.float32),
            jax.ShapeDtypeStruct((n, d), jnp.float32),
        ],
    )(p, dis, b1, g1, be1, hprev, ga, gb, gbias, w2)


def _tc_post(p, dis, b2, g2, be2, hprev, ga, gb, gbias, h_orig, res_w):
    n, d = p.shape[1], p.shape[2]

    def body(p_ref, dis_ref, b_ref, g_ref, be_ref, hp_ref, ga_ref, gb_ref,
             gbias_ref, ho_ref, rw_ref, dh_ref):
        hp = hp_ref[...]
        s = (p_ref[0] + p_ref[1]) * dis_ref[...] + b_ref[...]
        hn = _ln(s, g_ref[...], be_ref[...])
        z = (jnp.dot(hp, ga_ref[...], preferred_element_type=jnp.float32)
             + jnp.dot(hn, gb_ref[...], preferred_element_type=jnp.float32)
             + gbias_ref[...])
        gate = jax.nn.sigmoid(z)
        hg = gate * hn + (1.0 - gate) * hp
        dh_ref[...] = jnp.tanh(hg) + rw_ref[...] * ho_ref[...]

    row = lambda i: (i, 0)
    fix = lambda i: (0, 0)
    return pl.pallas_call(
        body,
        grid=(n // _BR,),
        in_specs=[
            pl.BlockSpec((2, _BR, d), lambda i: (0, i, 0)),
            pl.BlockSpec((_BR, 1), row),
            pl.BlockSpec((1, d), fix),
            pl.BlockSpec((1, d), fix),
            pl.BlockSpec((1, d), fix),
            pl.BlockSpec((_BR, d), row),
            pl.BlockSpec((d, d), fix),
            pl.BlockSpec((d, d), fix),
            pl.BlockSpec((1, d), fix),
            pl.BlockSpec((_BR, d), row),
            pl.BlockSpec((1, 1), fix),
        ],
        out_specs=pl.BlockSpec((_BR, d), row),
        out_shape=jax.ShapeDtypeStruct((n, d), jnp.float32),
    )(p, dis, b2, g2, be2, hprev, ga, gb, gbias, h_orig, res_w)


def kernel(t, h, edge_index, W0, b0, W1, b1, W2, b2, ln0_g, ln0_b, ln1_g,
           ln1_b, ln2_g, ln2_b, gate_W, gate_b, res_w):
    n, d = h.shape
    ei = edge_index.astype(jnp.int32)
    src = ei[0]
    dst = ei[1]
    e = src.shape[0]
    epw = _ECH * _EC
    padw = epw - e // _NW
    pad_dst = jnp.broadcast_to(n + jnp.arange(padw, dtype=jnp.int32) % 16,
                               (_NW, padw))
    src1 = jnp.concatenate(
        [src.reshape(_NW, e // _NW),
         jnp.zeros((_NW, padw), jnp.int32)], axis=1).reshape(-1)
    dst1 = jnp.concatenate(
        [dst.reshape(_NW, e // _NW), pad_dst], axis=1).reshape(-1)

    degp = _sc_degree(dst, n)
    degp3 = degp.reshape(_NC, n, 1)

    b0r = b0.reshape(1, d)
    b1r = b1.reshape(1, d)
    b2r = b2.reshape(1, d)
    g0 = ln0_g.reshape(1, d)
    be0 = ln0_b.reshape(1, d)
    g1 = ln1_g.reshape(1, d)
    be1 = ln1_b.reshape(1, d)
    g2 = ln2_g.reshape(1, d)
    be2 = ln2_b.reshape(1, d)
    ga = gate_W[:d]
    gb = gate_W[d:]
    gbias = gate_b.reshape(1, d)
    rw = res_w.reshape(1, 1)

    u0, dis = _tc_pre(h, W0, degp3)
    p0 = _sc_edge_sum(u0, src1, dst1)
    h1, u1 = _tc_mid0(p0, dis, b0r, g0, be0, W1)
    p1 = _sc_edge_sum(u1, src1, dst1)
    h2, u2 = _tc_mid1(p1, dis, b1r, g1, be1, h1, ga, gb, gbias, W2)
    p2 = _sc_edge_sum(u2, src1, dst1)
    dh = _tc_post(p2, dis, b2r, g2, be2, h2, ga, gb, gbias, h, rw)
    return dh


# speedup vs baseline: 3.3191x; 1.0000x over previous
import functools

import jax
import jax.numpy as jnp
from jax import lax
from jax.experimental import pallas as pl
from jax.experimental.pallas import tpu as pltpu
from jax.experimental.pallas import tpu_sc as plsc

_NC = 2
_NS = 16
_NW = _NC * _NS

_CHUNK = 80
_ZROWS = 128


def _sc_degree(dst, n_nodes):
    e = dst.shape[0]
    epw = e // _NW
    nch = epw // _CHUNK
    zlen = 640
    zstride = 624
    mesh = plsc.VectorSubcoreMesh(
        core_axis_name="c", subcore_axis_name="s",
        num_cores=_NC, num_subcores=_NS)

    @functools.partial(
        pl.kernel,
        out_type=jax.ShapeDtypeStruct((_NC * n_nodes,), jnp.float32),
        mesh=mesh,
        scratch_types=[
            pltpu.VMEM((_CHUNK,), jnp.int32),
            pltpu.VMEM((_CHUNK,), jnp.float32),
            pltpu.VMEM((zlen,), jnp.float32),
            pltpu.VMEM_SHARED((n_nodes,), jnp.float32),
        ],
    )
    def k(dst_hbm, ones_hbm, zeros_hbm, out_hbm, didx, ones_v, zb, acc):
        cid = lax.axis_index("c")
        sid = lax.axis_index("s")
        wid = cid * _NS + sid
        pltpu.sync_copy(ones_hbm, ones_v)
        pltpu.sync_copy(zeros_hbm, zb)
        pltpu.sync_copy(zb, acc.at[pl.ds(sid * zstride, zlen)])
        plsc.subcore_barrier()

        def body(g, _):
            off = wid * epw + g * _CHUNK
            pltpu.sync_copy(dst_hbm.at[pl.ds(off, _CHUNK)], didx)
            pltpu.sync_copy(ones_v, acc.at[didx], add=True)
            return 0

        lax.fori_loop(0, nch, body, 0)
        plsc.subcore_barrier()
        pltpu.sync_copy(acc.at[pl.ds(sid * zstride, zlen)], zb)
        pltpu.sync_copy(zb, out_hbm.at[pl.ds(cid * n_nodes + sid * zstride,
                                             zlen)])

    ones = jnp.ones((_CHUNK,), jnp.float32)
    zeros = jnp.zeros((zlen,), jnp.float32)
    return k(dst, ones, zeros).reshape(_NC, n_nodes)


_EC = 56
_ECH = 180
_RING = 6
_FIRE = 4
_LAG = 2
_IRING = 12
_IAHEAD = 6
_IUNROLL = 12
_RB = 80


def _sc_edge_sum(u, src1, dst1):
    n, d = u.shape
    nch = _ECH
    ec = _EC
    epw = nch * ec
    rstride = 624
    nz = 640 // _RB
    mesh = plsc.VectorSubcoreMesh(
        core_axis_name="c", subcore_axis_name="s",
        num_cores=_NC, num_subcores=_NS)

    @functools.partial(
        pl.kernel,
        out_type=jax.ShapeDtypeStruct((_NC, n, d), jnp.float32),
        mesh=mesh,
        scratch_types=[
            pltpu.VMEM((_IRING, ec), jnp.int32),
            pltpu.VMEM((_IRING, ec), jnp.int32),
            pltpu.VMEM((_RING, ec, d), jnp.float32),
            pltpu.VMEM_SHARED((n + 16, d), jnp.float32),
            pltpu.SemaphoreType.DMA,
            pltpu.SemaphoreType.DMA,
            pltpu.SemaphoreType.DMA,
        ],
    )
    def k(u_hbm, src_hbm, dst_hbm, zeros_hbm, out_hbm, sidx, didx, rows,
          acc, isem, gsem, ssem):
        cid = lax.axis_index("c")
        sid = lax.axis_index("s")
        wid = cid * _NS + sid
        zb = rows.at[0, pl.ds(0, _RB)]
        pltpu.sync_copy(zeros_hbm, zb)
        for j in range(nz):
            pltpu.sync_copy(zb, acc.at[pl.ds(sid * rstride + j * _RB, _RB)])

        def fire_idx(g, islot):
            off = wid * epw + g * ec
            pltpu.async_copy(src_hbm.at[pl.ds(off, ec)], sidx.at[islot],
                             isem)
            pltpu.async_copy(dst_hbm.at[pl.ds(off, ec)], didx.at[islot],
                             isem)

        def wait_idx():
            pltpu.make_async_copy(src_hbm.at[pl.ds(0, ec)], sidx.at[0],
                                  isem).wait()
            pltpu.make_async_copy(dst_hbm.at[pl.ds(0, ec)], didx.at[0],
                                  isem).wait()

        def fire_gather(islot, slot):
            pltpu.async_copy(u_hbm.at[sidx.at[islot]], rows.at[slot], gsem)

        def wait_gather(slot):
            pltpu.make_async_copy(u_hbm.at[sidx.at[0]], rows.at[slot],
                                  gsem).wait()

        def fire_scatter(islot, slot):
            pltpu.async_copy(rows.at[slot], acc.at[didx.at[islot]], ssem,
                             add=True)

        def wait_scatter(slot):
            pltpu.make_async_copy(rows.at[slot], acc.at[didx.at[0]],
                                  ssem).wait()

        plsc.subcore_barrier()
        for g in range(_IAHEAD):
            fire_idx(g, g)
        for g in range(_FIRE):
            wait_idx()
            fire_gather(g, g)

        def body(s, _):
            g0 = s * _IUNROLL
            for j in range(_IUNROLL):
                g = g0 + j
                r = j % _RING
                wait_gather(r)
                fire_scatter(j % _IRING, r)

                @pl.when(g >= _LAG)
                def _():
                    wait_scatter((r - _LAG) % _RING)

                @pl.when(g + _IAHEAD < nch)
                def _():
                    fire_idx(g + _IAHEAD, (j + _IAHEAD) % _IRING)

                @pl.when(g + _FIRE < nch)
                def _():
                    wait_idx()
                    fire_gather((j + _FIRE) % _IRING, (r + _FIRE) % _RING)

            return 0

        lax.fori_loop(0, nch // _IUNROLL, body, 0)
        for j in range(_LAG):
            wait_scatter((nch - _LAG + j) % _RING)
        plsc.subcore_barrier()
        for j in range(nz):
            r0 = sid * rstride + j * _RB
            pltpu.sync_copy(acc.at[pl.ds(r0, _RB)], zb)
            pltpu.sync_copy(zb, out_hbm.at[cid, pl.ds(r0, _RB)])

    zeros = jnp.zeros((_RB, d), jnp.float32)
    return k(u, src1, dst1, zeros)


_BR = 1000


def _tc_pre(h, w0, degp3):
    n, d = h.shape

    def body(h_ref, w_ref, dp_ref, u_ref, dis_ref):
        deg = dp_ref[0] + dp_ref[1]
        dis = jnp.where(deg > 0, lax.rsqrt(deg), 0.0)
        dis_ref[...] = dis
        u_ref[...] = jnp.dot(h_ref[...], w_ref[...],
                             preferred_element_type=jnp.float32) * dis

    return pl.pallas_call(
        body,
        grid=(n // _BR,),
        in_specs=[
            pl.BlockSpec((_BR, d), lambda i: (i, 0)),
            pl.BlockSpec((d, d), lambda i: (0, 0)),
            pl.BlockSpec((2, _BR, 1), lambda i: (0, i, 0)),
        ],
        out_specs=[
            pl.BlockSpec((_BR, d), lambda i: (i, 0)),
            pl.BlockSpec((_BR, 1), lambda i: (i, 0)),
        ],
        out_shape=[
            jax.ShapeDtypeStruct((n, d), jnp.float32),
            jax.ShapeDtypeStruct((n, 1), jnp.float32),
        ],
    )(h, w0, degp3)


def _ln(x, g, b):
    mu = jnp.mean(x, axis=-1, keepdims=True)
    xc = x - mu
    var = jnp.mean(xc * xc, axis=-1, keepdims=True)
    return xc * lax.rsqrt(var + 1e-5) * g + b


def _tc_mid0(p, dis, b0, g0, be0, w1):
    n, d = p.shape[1], p.shape[2]

    def body(p_ref, dis_ref, b_ref, g_ref, be_ref, w_ref, h_ref, u_ref):
        dis = dis_ref[...]
        s = (p_ref[0] + p_ref[1]) * dis + b_ref[...]
        hn = _ln(s, g_ref[...], be_ref[...])
        h_ref[...] = hn
        u_ref[...] = jnp.dot(hn, w_ref[...],
                             preferred_element_type=jnp.float32) * dis

    row = lambda i: (i, 0)
    fix = lambda i: (0, 0)
    return pl.pallas_call(
        body,
        grid=(n // _BR,),
        in_specs=[
            pl.BlockSpec((2, _BR, d), lambda i: (0, i, 0)),
            pl.BlockSpec((_BR, 1), row),
            pl.BlockSpec((1, d), fix),
            pl.BlockSpec((1, d), fix),
            pl.BlockSpec((1, d), fix),
            pl.BlockSpec((d, d), fix),
        ],
        out_specs=[pl.BlockSpec((_BR, d), row), pl.BlockSpec((_BR, d), row)],
        out_shape=[
            jax.ShapeDtypeStruct((n, d), jnp.float32),
            jax.ShapeDtypeStruct((n, d), jnp.float32),
        ],
    )(p, dis, b0, g0, be0, w1)


def _tc_mid1(p, dis, b1, g1, be1, hprev, ga, gb, gbias, w2):
    n, d = p.shape[1], p.shape[2]

    def body(p_ref, dis_ref, b_ref, g_ref, be_ref, hp_ref, ga_ref, gb_ref,
             gbias_ref, w_ref, h_ref, u_ref):
        dis = dis_ref[...]
        hp = hp_ref[...]
        s = (p_ref[0] + p_ref[1]) * dis + b_ref[...]
        hn = _ln(s, g_ref[...], be_ref[...])
        z = (jnp.dot(hp, ga_ref[...], preferred_element_type=jnp.float32)
             + jnp.dot(hn, gb_ref[...], preferred_element_type=jnp.float32)
             + gbias_ref[...])
        gate = jax.nn.sigmoid(z)
        hg = gate * hn + (1.0 - gate) * hp
        h_ref[...] = hg
        u_ref[...] = jnp.dot(hg, w_ref[...],
                             preferred_element_type=jnp.float32) * dis

    row = lambda i: (i, 0)
    fix = lambda i: (0, 0)
    return pl.pallas_call(
        body,
        grid=(n // _BR,),
        in_specs=[
            pl.BlockSpec((2, _BR, d), lambda i: (0, i, 0)),
            pl.BlockSpec((_BR, 1), row),
            pl.BlockSpec((1, d), fix),
            pl.BlockSpec((1, d), fix),
            pl.BlockSpec((1, d), fix),
            pl.BlockSpec((_BR, d), row),
            pl.BlockSpec((d, d), fix),
            pl.BlockSpec((d, d), fix),
            pl.BlockSpec((1, d), fix),
            pl.BlockSpec((d, d), fix),
        ],
        out_specs=[pl.BlockSpec((_BR, d), row), pl.BlockSpec((_BR, d), row)],
        out_shape=[
            jax.ShapeDtypeStruct((n, d), jnp.float32),
            jax.ShapeDtypeStruct((n, d), jnp.float32),
        ],
    )(p, dis, b1, g1, be1, hprev, ga, gb, gbias, w2)


def _tc_post(p, dis, b2, g2, be2, hprev, ga, gb, gbias, h_orig, res_w):
    n, d = p.shape[1], p.shape[2]

    def body(p_ref, dis_ref, b_ref, g_ref, be_ref, hp_ref, ga_ref, gb_ref,
             gbias_ref, ho_ref, rw_ref, dh_ref):
        hp = hp_ref[...]
        s = (p_ref[0] + p_ref[1]) * dis_ref[...] + b_ref[...]
        hn = _ln(s, g_ref[...], be_ref[...])
        z = (jnp.dot(hp, ga_ref[...], preferred_element_type=jnp.float32)
             + jnp.dot(hn, gb_ref[...], preferred_element_type=jnp.float32)
             + gbias_ref[...])
        gate = jax.nn.sigmoid(z)
        hg = gate * hn + (1.0 - gate) * hp
        dh_ref[...] = jnp.tanh(hg) + rw_ref[...] * ho_ref[...]

    row = lambda i: (i, 0)
    fix = lambda i: (0, 0)
    return pl.pallas_call(
        body,
        grid=(n // _BR,),
        in_specs=[
            pl.BlockSpec((2, _BR, d), lambda i: (0, i, 0)),
            pl.BlockSpec((_BR, 1), row),
            pl.BlockSpec((1, d), fix),
            pl.BlockSpec((1, d), fix),
            pl.BlockSpec((1, d), fix),
            pl.BlockSpec((_BR, d), row),
            pl.BlockSpec((d, d), fix),
            pl.BlockSpec((d, d), fix),
            pl.BlockSpec((1, d), fix),
            pl.BlockSpec((_BR, d), row),
            pl.BlockSpec((1, 1), fix),
        ],
        out_specs=pl.BlockSpec((_BR, d), row),
        out_shape=jax.ShapeDtypeStruct((n, d), jnp.float32),
    )(p, dis, b2, g2, be2, hprev, ga, gb, gbias, h_orig, res_w)


def kernel(t, h, edge_index, W0, b0, W1, b1, W2, b2, ln0_g, ln0_b, ln1_g,
           ln1_b, ln2_g, ln2_b, gate_W, gate_b, res_w):
    n, d = h.shape
    ei = edge_index.astype(jnp.int32)
    src = ei[0]
    dst = ei[1]
    e = src.shape[0]
    epw = _ECH * _EC
    padw = epw - e // _NW
    pad_dst = jnp.broadcast_to(n + jnp.arange(padw, dtype=jnp.int32) % 16,
                               (_NW, padw))
    src1 = jnp.concatenate(
        [src.reshape(_NW, e // _NW),
         jnp.zeros((_NW, padw), jnp.int32)], axis=1).reshape(-1)
    dst1 = jnp.concatenate(
        [dst.reshape(_NW, e // _NW), pad_dst], axis=1).reshape(-1)

    degp = _sc_degree(dst, n)
    degp3 = degp.reshape(_NC, n, 1)

    b0r = b0.reshape(1, d)
    b1r = b1.reshape(1, d)
    b2r = b2.reshape(1, d)
    g0 = ln0_g.reshape(1, d)
    be0 = ln0_b.reshape(1, d)
    g1 = ln1_g.reshape(1, d)
    be1 = ln1_b.reshape(1, d)
    g2 = ln2_g.reshape(1, d)
    be2 = ln2_b.reshape(1, d)
    ga = gate_W[:d]
    gb = gate_W[d:]
    gbias = gate_b.reshape(1, d)
    rw = res_w.reshape(1, 1)

    u0, dis = _tc_pre(h, W0, degp3)
    p0 = _sc_edge_sum(u0, src1, dst1)
    h1, u1 = _tc_mid0(p0, dis, b0r, g0, be0, W1)
    p1 = _sc_edge_sum(u1, src1, dst1)
    h2, u2 = _tc_mid1(p1, dis, b1r, g1, be1, h1, ga, gb, gbias, W2)
    p2 = _sc_edge_sum(u2, src1, dst1)
    dh = _tc_post(p2, dis, b2r, g2, be2, h2, ga, gb, gbias, h, rw)
    return dh


# speedup vs baseline: 3.4616x; 1.0429x over previous
import functools

import jax
import jax.numpy as jnp
from jax import lax
from jax.experimental import pallas as pl
from jax.experimental.pallas import tpu as pltpu
from jax.experimental.pallas import tpu_sc as plsc

_NC = 2
_NS = 16
_NW = _NC * _NS

_CHUNK = 80
_ZROWS = 128


_DC = 72
_DIR = 4


def _sc_degree(dst1, n_nodes):
    e = dst1.shape[0]
    epw = e // _NW
    nch = epw // _DC
    na = n_nodes + 16
    zlen = 640
    zstride = 624
    mesh = plsc.VectorSubcoreMesh(
        core_axis_name="c", subcore_axis_name="s",
        num_cores=_NC, num_subcores=_NS)

    @functools.partial(
        pl.kernel,
        out_type=jax.ShapeDtypeStruct((_NC * na,), jnp.float32),
        mesh=mesh,
        scratch_types=[
            pltpu.VMEM((_DIR, _DC), jnp.int32),
            pltpu.VMEM((_DC,), jnp.float32),
            pltpu.VMEM((zlen,), jnp.float32),
            pltpu.VMEM_SHARED((na,), jnp.float32),
            pltpu.SemaphoreType.DMA,
            pltpu.SemaphoreType.DMA,
        ],
    )
    def k(dst_hbm, ones_hbm, zeros_hbm, out_hbm, didx, ones_v, zb, acc,
          isem, ssem):
        cid = lax.axis_index("c")
        sid = lax.axis_index("s")
        wid = cid * _NS + sid
        pltpu.sync_copy(ones_hbm, ones_v)
        pltpu.sync_copy(zeros_hbm, zb)
        pltpu.sync_copy(zb, acc.at[pl.ds(sid * zstride, zlen)])

        def fire_idx(g, islot):
            pltpu.async_copy(dst_hbm.at[pl.ds(wid * epw + g * _DC, _DC)],
                             didx.at[islot], isem)

        def wait_idx():
            pltpu.make_async_copy(dst_hbm.at[pl.ds(0, _DC)], didx.at[0],
                                  isem).wait()

        def fire_scatter(islot):
            pltpu.async_copy(ones_v, acc.at[didx.at[islot]], ssem, add=True)

        def wait_scatter():
            pltpu.make_async_copy(ones_v, acc.at[didx.at[0]], ssem).wait()

        plsc.subcore_barrier()
        fire_idx(0, 0)
        fire_idx(1, 1)

        def body(s, _):
            g0 = s * _DIR
            for j in range(_DIR):
                g = g0 + j
                wait_idx()
                fire_scatter(j)

                @pl.when(g >= 2)
                def _():
                    wait_scatter()

                @pl.when(g + 2 < nch)
                def _():
                    fire_idx(g + 2, (j + 2) % _DIR)

            return 0

        lax.fori_loop(0, nch // _DIR, body, 0)
        wait_scatter()
        wait_scatter()
        plsc.subcore_barrier()
        pltpu.sync_copy(acc.at[pl.ds(sid * zstride, zlen)], zb)
        pltpu.sync_copy(zb, out_hbm.at[pl.ds(cid * na + sid * zstride,
                                             zlen)])

    ones = jnp.ones((_DC,), jnp.float32)
    zeros = jnp.zeros((zlen,), jnp.float32)
    return k(dst1, ones, zeros).reshape(_NC, na)[:, :n_nodes]


_EC = 56
_ECH = 180
_RING = 6
_FIRE = 4
_LAG = 2
_IRING = 12
_IAHEAD = 6
_IUNROLL = 12
_RB = 80


def _sc_edge_sum(u, src1, dst1):
    n, d = u.shape
    nch = _ECH
    ec = _EC
    epw = nch * ec
    rstride = 624
    nz = 640 // _RB
    mesh = plsc.VectorSubcoreMesh(
        core_axis_name="c", subcore_axis_name="s",
        num_cores=_NC, num_subcores=_NS)

    @functools.partial(
        pl.kernel,
        out_type=jax.ShapeDtypeStruct((_NC, n, d), jnp.float32),
        mesh=mesh,
        scratch_types=[
            pltpu.VMEM((_IRING, ec), jnp.int32),
            pltpu.VMEM((_IRING, ec), jnp.int32),
            pltpu.VMEM((_RING, ec, d), jnp.float32),
            pltpu.VMEM_SHARED((n + 16, d), jnp.float32),
            pltpu.SemaphoreType.DMA,
            pltpu.SemaphoreType.DMA,
            pltpu.SemaphoreType.DMA,
        ],
    )
    def k(u_hbm, src_hbm, dst_hbm, zeros_hbm, out_hbm, sidx, didx, rows,
          acc, isem, gsem, ssem):
        cid = lax.axis_index("c")
        sid = lax.axis_index("s")
        wid = cid * _NS + sid
        zb = rows.at[0, pl.ds(0, _RB)]
        pltpu.sync_copy(zeros_hbm, zb)
        for j in range(nz):
            pltpu.sync_copy(zb, acc.at[pl.ds(sid * rstride + j * _RB, _RB)])

        def fire_idx(g, islot):
            off = wid * epw + g * ec
            pltpu.async_copy(src_hbm.at[pl.ds(off, ec)], sidx.at[islot],
                             isem)
            pltpu.async_copy(dst_hbm.at[pl.ds(off, ec)], didx.at[islot],
                             isem)

        def wait_idx():
            pltpu.make_async_copy(src_hbm.at[pl.ds(0, ec)], sidx.at[0],
                                  isem).wait()
            pltpu.make_async_copy(dst_hbm.at[pl.ds(0, ec)], didx.at[0],
                                  isem).wait()

        def fire_gather(islot, slot):
            pltpu.async_copy(u_hbm.at[sidx.at[islot]], rows.at[slot], gsem)

        def wait_gather(slot):
            pltpu.make_async_copy(u_hbm.at[sidx.at[0]], rows.at[slot],
                                  gsem).wait()

        def fire_scatter(islot, slot):
            pltpu.async_copy(rows.at[slot], acc.at[didx.at[islot]], ssem,
                             add=True)

        def wait_scatter(slot):
            pltpu.make_async_copy(rows.at[slot], acc.at[didx.at[0]],
                                  ssem).wait()

        plsc.subcore_barrier()
        for g in range(_IAHEAD):
            fire_idx(g, g)
        for g in range(_FIRE):
            wait_idx()
            fire_gather(g, g)

        def body(s, _):
            g0 = s * _IUNROLL
            for j in range(_IUNROLL):
                g = g0 + j
                r = j % _RING
                wait_gather(r)
                fire_scatter(j % _IRING, r)

                @pl.when(g >= _LAG)
                def _():
                    wait_scatter((r - _LAG) % _RING)

                @pl.when(g + _IAHEAD < nch)
                def _():
                    fire_idx(g + _IAHEAD, (j + _IAHEAD) % _IRING)

                @pl.when(g + _FIRE < nch)
                def _():
                    wait_idx()
                    fire_gather((j + _FIRE) % _IRING, (r + _FIRE) % _RING)

            return 0

        lax.fori_loop(0, nch // _IUNROLL, body, 0)
        for j in range(_LAG):
            wait_scatter((nch - _LAG + j) % _RING)
        plsc.subcore_barrier()
        for j in range(nz):
            r0 = sid * rstride + j * _RB
            pltpu.sync_copy(acc.at[pl.ds(r0, _RB)], zb)
            pltpu.sync_copy(zb, out_hbm.at[cid, pl.ds(r0, _RB)])

    zeros = jnp.zeros((_RB, d), jnp.float32)
    return k(u, src1, dst1, zeros)


_BR = 1000


def _tc_pre(h, w0, degp3):
    n, d = h.shape

    def body(h_ref, w_ref, dp_ref, u_ref, dis_ref):
        deg = dp_ref[0] + dp_ref[1]
        dis = jnp.where(deg > 0, lax.rsqrt(deg), 0.0)
        dis_ref[...] = dis
        u_ref[...] = jnp.dot(h_ref[...], w_ref[...],
                             preferred_element_type=jnp.float32) * dis

    return pl.pallas_call(
        body,
        grid=(n // _BR,),
        in_specs=[
            pl.BlockSpec((_BR, d), lambda i: (i, 0)),
            pl.BlockSpec((d, d), lambda i: (0, 0)),
            pl.BlockSpec((2, _BR, 1), lambda i: (0, i, 0)),
        ],
        out_specs=[
            pl.BlockSpec((_BR, d), lambda i: (i, 0)),
            pl.BlockSpec((_BR, 1), lambda i: (i, 0)),
        ],
        out_shape=[
            jax.ShapeDtypeStruct((n, d), jnp.float32),
            jax.ShapeDtypeStruct((n, 1), jnp.float32),
        ],
    )(h, w0, degp3)


def _ln(x, g, b):
    mu = jnp.mean(x, axis=-1, keepdims=True)
    xc = x - mu
    var = jnp.mean(xc * xc, axis=-1, keepdims=True)
    return xc * lax.rsqrt(var + 1e-5) * g + b


def _tc_mid0(p, dis, b0, g0, be0, w1):
    n, d = p.shape[1], p.shape[2]

    def body(p_ref, dis_ref, b_ref, g_ref, be_ref, w_ref, h_ref, u_ref):
        dis = dis_ref[...]
        s = (p_ref[0] + p_ref[1]) * dis + b_ref[...]
        hn = _ln(s, g_ref[...], be_ref[...])
        h_ref[...] = hn
        u_ref[...] = jnp.dot(hn, w_ref[...],
                             preferred_element_type=jnp.float32) * dis

    row = lambda i: (i, 0)
    fix = lambda i: (0, 0)
    return pl.pallas_call(
        body,
        grid=(n // _BR,),
        in_specs=[
            pl.BlockSpec((2, _BR, d), lambda i: (0, i, 0)),
            pl.BlockSpec((_BR, 1), row),
            pl.BlockSpec((1, d), fix),
            pl.BlockSpec((1, d), fix),
            pl.BlockSpec((1, d), fix),
            pl.BlockSpec((d, d), fix),
        ],
        out_specs=[pl.BlockSpec((_BR, d), row), pl.BlockSpec((_BR, d), row)],
        out_shape=[
            jax.ShapeDtypeStruct((n, d), jnp.float32),
            jax.ShapeDtypeStruct((n, d), jnp.float32),
        ],
    )(p, dis, b0, g0, be0, w1)


def _tc_mid1(p, dis, b1, g1, be1, hprev, ga, gb, gbias, w2):
    n, d = p.shape[1], p.shape[2]

    def body(p_ref, dis_ref, b_ref, g_ref, be_ref, hp_ref, ga_ref, gb_ref,
             gbias_ref, w_ref, h_ref, u_ref):
        dis = dis_ref[...]
        hp = hp_ref[...]
        s = (p_ref[0] + p_ref[1]) * dis + b_ref[...]
        hn = _ln(s, g_ref[...], be_ref[...])
        z = (jnp.dot(hp, ga_ref[...], preferred_element_type=jnp.float32)
             + jnp.dot(hn, gb_ref[...], preferred_element_type=jnp.float32)
             + gbias_ref[...])
        gate = jax.nn.sigmoid(z)
        hg = gate * hn + (1.0 - gate) * hp
        h_ref[...] = hg
        u_ref[...] = jnp.dot(hg, w_ref[...],
                             preferred_element_type=jnp.float32) * dis

    row = lambda i: (i, 0)
    fix = lambda i: (0, 0)
    return pl.pallas_call(
        body,
        grid=(n // _BR,),
        in_specs=[
            pl.BlockSpec((2, _BR, d), lambda i: (0, i, 0)),
            pl.BlockSpec((_BR, 1), row),
            pl.BlockSpec((1, d), fix),
            pl.BlockSpec((1, d), fix),
            pl.BlockSpec((1, d), fix),
            pl.BlockSpec((_BR, d), row),
            pl.BlockSpec((d, d), fix),
            pl.BlockSpec((d, d), fix),
            pl.BlockSpec((1, d), fix),
            pl.BlockSpec((d, d), fix),
        ],
        out_specs=[pl.BlockSpec((_BR, d), row), pl.BlockSpec((_BR, d), row)],
        out_shape=[
            jax.ShapeDtypeStruct((n, d), jnp.float32),
            jax.ShapeDtypeStruct((n, d), jnp.float32),
        ],
    )(p, dis, b1, g1, be1, hprev, ga, gb, gbias, w2)


def _tc_post(p, dis, b2, g2, be2, hprev, ga, gb, gbias, h_orig, res_w):
    n, d = p.shape[1], p.shape[2]

    def body(p_ref, dis_ref, b_ref, g_ref, be_ref, hp_ref, ga_ref, gb_ref,
             gbias_ref, ho_ref, rw_ref, dh_ref):
        hp = hp_ref[...]
        s = (p_ref[0] + p_ref[1]) * dis_ref[...] + b_ref[...]
        hn = _ln(s, g_ref[...], be_ref[...])
        z = (jnp.dot(hp, ga_ref[...], preferred_element_type=jnp.float32)
             + jnp.dot(hn, gb_ref[...], preferred_element_type=jnp.float32)
             + gbias_ref[...])
        gate = jax.nn.sigmoid(z)
        hg = gate * hn + (1.0 - gate) * hp
        dh_ref[...] = jnp.tanh(hg) + rw_ref[...] * ho_ref[...]

    row = lambda i: (i, 0)
    fix = lambda i: (0, 0)
    return pl.pallas_call(
        body,
        grid=(n // _BR,),
        in_specs=[
            pl.BlockSpec((2, _BR, d), lambda i: (0, i, 0)),
            pl.BlockSpec((_BR, 1), row),
            pl.BlockSpec((1, d), fix),
            pl.BlockSpec((1, d), fix),
            pl.BlockSpec((1, d), fix),
            pl.BlockSpec((_BR, d), row),
            pl.BlockSpec((d, d), fix),
            pl.BlockSpec((d, d), fix),
            pl.BlockSpec((1, d), fix),
            pl.BlockSpec((_BR, d), row),
            pl.BlockSpec((1, 1), fix),
        ],
        out_specs=pl.BlockSpec((_BR, d), row),
        out_shape=jax.ShapeDtypeStruct((n, d), jnp.float32),
    )(p, dis, b2, g2, be2, hprev, ga, gb, gbias, h_orig, res_w)


def kernel(t, h, edge_index, W0, b0, W1, b1, W2, b2, ln0_g, ln0_b, ln1_g,
           ln1_b, ln2_g, ln2_b, gate_W, gate_b, res_w):
    n, d = h.shape
    ei = edge_index.astype(jnp.int32)
    src = ei[0]
    dst = ei[1]
    e = src.shape[0]
    epw = _ECH * _EC
    padw = epw - e // _NW
    pad_dst = jnp.broadcast_to(n + jnp.arange(padw, dtype=jnp.int32) % 16,
                               (_NW, padw))
    src1 = jnp.concatenate(
        [src.reshape(_NW, e // _NW),
         jnp.zeros((_NW, padw), jnp.int32)], axis=1).reshape(-1)
    dst1 = jnp.concatenate(
        [dst.reshape(_NW, e // _NW), pad_dst], axis=1).reshape(-1)

    degp = _sc_degree(dst1, n)
    degp3 = degp.reshape(_NC, n, 1)

    b0r = b0.reshape(1, d)
    b1r = b1.reshape(1, d)
    b2r = b2.reshape(1, d)
    g0 = ln0_g.reshape(1, d)
    be0 = ln0_b.reshape(1, d)
    g1 = ln1_g.reshape(1, d)
    be1 = ln1_b.reshape(1, d)
    g2 = ln2_g.reshape(1, d)
    be2 = ln2_b.reshape(1, d)
    ga = gate_W[:d]
    gb = gate_W[d:]
    gbias = gate_b.reshape(1, d)
    rw = res_w.reshape(1, 1)

    u0, dis = _tc_pre(h, W0, degp3)
    p0 = _sc_edge_sum(u0, src1, dst1)
    h1, u1 = _tc_mid0(p0, dis, b0r, g0, be0, W1)
    p1 = _sc_edge_sum(u1, src1, dst1)
    h2, u2 = _tc_mid1(p1, dis, b1r, g1, be1, h1, ga, gb, gbias, W2)
    p2 = _sc_edge_sum(u2, src1, dst1)
    dh = _tc_post(p2, dis, b2r, g2, be2, h2, ga, gb, gbias, h, rw)
    return dh
